# baseline (speedup 1.0000x reference)
import dataclasses
import functools

import jax
import jax.numpy as jnp
from jax import lax
from jax.experimental import pallas as pl
from jax.experimental.pallas import tpu as pltpu
from jax.experimental.pallas import tpu_sc as plsc

IN_DIM = 4096
HID = 4096
FS = IN_DIM * HID
NSUB = 16
NCORE = 2
NW = NSUB * NCORE
SHARD = 81920
NSH = -(-FS // SHARD)
NGEN = -(-NSH // NW)
NSHP = NGEN * NW
OTBL = 208
CHUNK = 1024
WIN = 512
WSH = 9


def _mesh_and_params():
    mesh = plsc.VectorSubcoreMesh(core_axis_name="c", subcore_axis_name="s")
    cp = pltpu.CompilerParams()
    if "needs_layout_passes" in pltpu.CompilerParams.__dataclass_fields__:
        cp = dataclasses.replace(cp, needs_layout_passes=False)
    return mesh, cp


def _bin_phase(rows_p, cols_p, vals_p):
    nnzp = rows_p.shape[0]
    share = nnzp // NW
    n_pairs = share // (2 * CHUNK)
    scap = (share // WIN + 1) * WIN
    mesh, cp = _mesh_and_params()

    @functools.partial(
        pl.kernel,
        compiler_params=cp,
        out_type=(
            jax.ShapeDtypeStruct((NW, scap), jnp.int32),
            jax.ShapeDtypeStruct((NW, scap), jnp.float32),
            jax.ShapeDtypeStruct((NW, OTBL), jnp.int32),
        ),
        mesh=mesh,
        scratch_types=[
            pltpu.VMEM((CHUNK,), jnp.int32),
            pltpu.VMEM((CHUNK,), jnp.int32),
            pltpu.VMEM((CHUNK,), jnp.float32),
            pltpu.VMEM((CHUNK,), jnp.int32),
            pltpu.VMEM((CHUNK,), jnp.int32),
            pltpu.VMEM((CHUNK,), jnp.float32),
            pltpu.VMEM((OTBL,), jnp.int32),
            pltpu.VMEM((OTBL,), jnp.int32),
            pltpu.VMEM((OTBL,), jnp.int32),
            pltpu.VMEM((scap,), jnp.int32),
            pltpu.VMEM((scap,), jnp.float32),
            pltpu.SemaphoreType.DMA,
            pltpu.SemaphoreType.DMA,
        ],
    )
    def bin_kernel(rows_hbm, cols_hbm, vals_hbm,
                   sidx_hbm, sval_hbm, offs_hbm,
                   r_a, c_a, v_a, r_b, c_b, v_b,
                   cnts, offs, curs, sidx_v, sval_v,
                   sem_a, sem_b):
        c = lax.axis_index("c")
        s = lax.axis_index("s")
        w = s * NCORE + c
        my_off = w * share

        zero16 = jnp.zeros((16,), jnp.int32)

        @pl.loop(0, OTBL // 16)
        def _z(g):
            cnts[pl.ds(g * 16, 16)] = zero16

        def load_chunk(ci, r_v, c_v, v_v, sem, with_vals):
            off = my_off + ci * CHUNK
            pltpu.async_copy(rows_hbm.at[pl.ds(off, CHUNK)], r_v, sem)
            pltpu.async_copy(cols_hbm.at[pl.ds(off, CHUNK)], c_v, sem)
            if with_vals:
                pltpu.async_copy(vals_hbm.at[pl.ds(off, CHUNK)], v_v, sem)

        def wait_inputs(r_v, sem, n):
            for _ in range(n):
                pltpu.make_async_copy(
                    rows_hbm.at[pl.ds(0, CHUNK)], r_v, sem).wait()

        def count_chunk(r_v, c_v):
            @pl.loop(0, CHUNK // 64)
            def _b(it):
                for u in range(4):
                    g = it * 64 + u * 16
                    rv = r_v[pl.ds(g, 16)]
                    cv = c_v[pl.ds(g, 16)]
                    t = (rv * HID + cv) // SHARD
                    cnt, lastm = plsc.scan_count(t)
                    cur = plsc.load_gather(cnts, [t])
                    plsc.store_scatter(cnts, [t], cur + cnt, mask=lastm)

        def place_chunk(r_v, c_v, v_v):
            @pl.loop(0, CHUNK // 64)
            def _b(it):
                for u in range(4):
                    g = it * 64 + u * 16
                    rv = r_v[pl.ds(g, 16)]
                    cv = c_v[pl.ds(g, 16)]
                    vv = v_v[pl.ds(g, 16)]
                    flat = rv * HID + cv
                    t = flat // SHARD
                    lidx = flat - t * SHARD
                    cnt, lastm = plsc.scan_count(t)
                    cur = plsc.load_gather(curs, [t])
                    pos = cur + cnt - 1
                    plsc.store_scatter(sidx_v, [pos], lidx)
                    plsc.store_scatter(sval_v, [pos], vv)
                    plsc.store_scatter(curs, [t], cur + cnt, mask=lastm)

        load_chunk(0, r_a, c_a, v_a, sem_a, False)
        load_chunk(1, r_b, c_b, v_b, sem_b, False)

        @pl.loop(0, n_pairs)
        def _pa(i):
            wait_inputs(r_a, sem_a, 2)
            count_chunk(r_a, c_a)

            @pl.when(i < n_pairs - 1)
            def _():
                load_chunk(2 * i + 2, r_a, c_a, v_a, sem_a, False)

            wait_inputs(r_b, sem_b, 2)
            count_chunk(r_b, c_b)

            @pl.when(i < n_pairs - 1)
            def _():
                load_chunk(2 * i + 3, r_b, c_b, v_b, sem_b, False)

        def pfx_body(g, carry):
            cv = cnts[pl.ds(g * 16, 16)]
            inc = plsc.cumsum(cv)
            ex = inc - cv + carry
            offs[pl.ds(g * 16, 16)] = ex
            curs[pl.ds(g * 16, 16)] = ex
            return carry + jnp.max(inc)

        lax.fori_loop(0, OTBL // 16, pfx_body, 0)
        pltpu.sync_copy(offs, offs_hbm.at[w])

        load_chunk(0, r_a, c_a, v_a, sem_a, True)
        load_chunk(1, r_b, c_b, v_b, sem_b, True)

        @pl.loop(0, n_pairs)
        def _pb(i):
            wait_inputs(r_a, sem_a, 3)
            place_chunk(r_a, c_a, v_a)

            @pl.when(i < n_pairs - 1)
            def _():
                load_chunk(2 * i + 2, r_a, c_a, v_a, sem_a, True)

            wait_inputs(r_b, sem_b, 3)
            place_chunk(r_b, c_b, v_b)

            @pl.when(i < n_pairs - 1)
            def _():
                load_chunk(2 * i + 3, r_b, c_b, v_b, sem_b, True)

        pltpu.sync_copy(sidx_v, sidx_hbm.at[w])
        pltpu.sync_copy(sval_v, sval_hbm.at[w])

    return bin_kernel(rows_p, cols_p, vals_p)


def _accum_phase(sidx, sval, offs):
    mesh, cp = _mesh_and_params()

    @functools.partial(
        pl.kernel,
        compiler_params=cp,
        out_type=jax.ShapeDtypeStruct((NSHP * SHARD,), jnp.float32),
        mesh=mesh,
        scratch_types=[
            pltpu.VMEM((SHARD,), jnp.float32),
            pltpu.VMEM((NW, WIN), jnp.int32),
            pltpu.VMEM((NW, WIN), jnp.float32),
            pltpu.VMEM((NW, OTBL), jnp.int32),
            pltpu.SemaphoreType.DMA,
        ],
    )
    def accum_kernel(sidx_hbm, sval_hbm, offs_hbm, w_hbm,
                     acc, widx, wval, offs_all, sem):
        c = lax.axis_index("c")
        s = lax.axis_index("s")
        w = s * NCORE + c

        pltpu.sync_copy(offs_hbm, offs_all)
        zero16f = jnp.zeros((16,), jnp.float32)
        lanes = jnp.arange(16, dtype=jnp.int32)
        imin = jnp.int32(-(2 ** 31))

        def tbl_at(p, j):
            jal = pl.multiple_of(j & ~15, 8)
            vec = offs_all[p, pl.ds(jal, 16)]
            return jnp.max(jnp.where(lanes == j - jal, vec, imin))

        def seg_bounds(p, shc):
            return tbl_at(p, shc), tbl_at(p, shc + 1)

        @pl.loop(0, NGEN)
        def _gen(g):
            sh = g * NW + w
            shc = jnp.minimum(sh, OTBL - 2)

            @pl.loop(0, SHARD // 256)
            def _zb(j):
                for l in range(16):
                    acc[pl.ds(j * 256 + l * 16, 16)] = zero16f

            @pl.loop(0, NW)
            def _iss(p):
                st, _ = seg_bounds(p, shc)
                sal = pl.multiple_of(st & ~(WIN - 1), WIN)
                pltpu.async_copy(sidx_hbm.at[p, pl.ds(sal, WIN)],
                                 widx.at[p], sem)
                pltpu.async_copy(sval_hbm.at[p, pl.ds(sal, WIN)],
                                 wval.at[p], sem)

            @pl.loop(0, NW)
            def _drn(p):
                pltpu.make_async_copy(
                    sidx_hbm.at[0, pl.ds(0, WIN)], widx.at[p], sem).wait()
                pltpu.make_async_copy(
                    sval_hbm.at[0, pl.ds(0, WIN)], wval.at[p], sem).wait()

            @pl.loop(0, NW)
            def _proc(p):
                st, en = seg_bounds(p, shc)
                sal = pl.multiple_of(st & ~(WIN - 1), WIN)

                @pl.when(jnp.logical_and(en > st, sh < NSH))
                def _():
                    @pl.loop(0, WIN // 16)
                    def _g(gg):
                        posv = sal + gg * 16 + lanes
                        m = jnp.logical_and(posv >= st, posv < en)
                        lidx = widx[p, pl.ds(gg * 16, 16)]
                        vv = wval[p, pl.ds(gg * 16, 16)]
                        plsc.addupdate_scatter(acc, [lidx], vv, mask=m)

                    def rem_body(wi, carry):
                        wo = pl.multiple_of(wi * WIN, WIN)
                        pltpu.sync_copy(
                            sidx_hbm.at[p, pl.ds(wo, WIN)],
                            widx.at[p])
                        pltpu.sync_copy(
                            sval_hbm.at[p, pl.ds(wo, WIN)],
                            wval.at[p])

                        @pl.loop(0, WIN // 16)
                        def _g2(gg):
                            posv = wi * WIN + gg * 16 + lanes
                            m = posv < en
                            lidx = widx[p, pl.ds(gg * 16, 16)]
                            vv = wval[p, pl.ds(gg * 16, 16)]
                            plsc.addupdate_scatter(acc, [lidx], vv,
                                                   mask=m)

                        return carry

                    lax.fori_loop(
                        lax.shift_right_logical(sal, WSH) + 1,
                        lax.shift_right_logical(en + WIN - 1, WSH),
                        rem_body, 0)

            pltpu.sync_copy(
                acc, w_hbm.at[pl.ds(pl.multiple_of(sh * SHARD, 8), SHARD)])

    return accum_kernel(sidx, sval, offs)


def _mm_body(x_ref, w_ref, b_ref, o_ref):
    xb = x_ref[...]
    wb = w_ref[...].astype(jnp.bfloat16)
    acc = jnp.dot(xb, wb, preferred_element_type=jnp.float32)
    o_ref[...] = jnp.tanh(acc + b_ref[...])


def _matmul(xb, w, bias2d):
    batch = xb.shape[0]
    bn = 512
    return pl.pallas_call(
        _mm_body,
        grid=(HID // bn,),
        in_specs=[
            pl.BlockSpec((batch, IN_DIM), lambda j: (0, 0)),
            pl.BlockSpec((IN_DIM, bn), lambda j: (0, j)),
            pl.BlockSpec((1, bn), lambda j: (0, j)),
        ],
        out_specs=pl.BlockSpec((batch, bn), lambda j: (0, j)),
        out_shape=jax.ShapeDtypeStruct((batch, HID), jnp.float32),
    )(xb, w, bias2d)


def kernel(x, values, bias, rows, cols):
    nnz = rows.shape[0]
    grain = NW * 2 * CHUNK
    nnzp = -(-nnz // grain) * grain
    pad = nnzp - nnz
    rows_p = jnp.concatenate(
        [rows, jnp.full((pad,), IN_DIM, jnp.int32)])
    cols_p = jnp.concatenate([cols, jnp.zeros((pad,), jnp.int32)])
    vals_p = jnp.concatenate([values, jnp.zeros((pad,), jnp.float32)])

    sidx, sval, offs = _bin_phase(rows_p, cols_p, vals_p)
    w_flat = _accum_phase(sidx, sval, offs)
    w = w_flat[:FS].reshape(IN_DIM, HID)
    xb = x.astype(jnp.bfloat16)
    return _matmul(xb, w, bias.reshape(1, HID))


# speedup vs baseline: 1.3551x; 1.3551x over previous
import dataclasses
import functools

import jax
import jax.numpy as jnp
from jax import lax
from jax.experimental import pallas as pl
from jax.experimental.pallas import tpu as pltpu
from jax.experimental.pallas import tpu_sc as plsc

IN_DIM = 4096
HID = 4096
FS = IN_DIM * HID
NSUB = 16
NCORE = 2
NW = NSUB * NCORE
SHARD = 65536
SSH = 16
NSH = FS // SHARD
NGEN = NSH // NW
OTBL = 272
NLANE = 4
CHUNK = 1024
WIN = 512


def _mesh_and_params():
    mesh = plsc.VectorSubcoreMesh(core_axis_name="c", subcore_axis_name="s")
    cp = pltpu.CompilerParams()
    if "needs_layout_passes" in pltpu.CompilerParams.__dataclass_fields__:
        cp = dataclasses.replace(cp, needs_layout_passes=False)
    return mesh, cp


def _bin_phase(rows_p, cols_p, vals_p):
    nnzp = rows_p.shape[0]
    share = nnzp // NW
    n_pairs = share // (2 * CHUNK)
    scap = (share // WIN + 1) * WIN
    mesh, cp = _mesh_and_params()

    @functools.partial(
        pl.kernel,
        compiler_params=cp,
        out_type=(
            jax.ShapeDtypeStruct((NW, scap), jnp.int32),
            jax.ShapeDtypeStruct((NW, scap), jnp.float32),
            jax.ShapeDtypeStruct((NW, OTBL), jnp.int32),
        ),
        mesh=mesh,
        scratch_types=[
            pltpu.VMEM((CHUNK,), jnp.int32),
            pltpu.VMEM((CHUNK,), jnp.int32),
            pltpu.VMEM((CHUNK,), jnp.float32),
            pltpu.VMEM((CHUNK,), jnp.int32),
            pltpu.VMEM((CHUNK,), jnp.int32),
            pltpu.VMEM((CHUNK,), jnp.float32),
            pltpu.VMEM((OTBL,), jnp.int32),
            pltpu.VMEM((OTBL,), jnp.int32),
            pltpu.VMEM((OTBL,), jnp.int32),
            pltpu.VMEM((OTBL,), jnp.int32),
            pltpu.VMEM((OTBL,), jnp.int32),
            pltpu.VMEM((OTBL,), jnp.int32),
            pltpu.VMEM((OTBL,), jnp.int32),
            pltpu.VMEM((OTBL,), jnp.int32),
            pltpu.VMEM((OTBL,), jnp.int32),
            pltpu.VMEM((scap,), jnp.int32),
            pltpu.VMEM((scap,), jnp.float32),
            pltpu.SemaphoreType.DMA,
            pltpu.SemaphoreType.DMA,
        ],
    )
    def bin_kernel(rows_hbm, cols_hbm, vals_hbm,
                   sidx_hbm, sval_hbm, offs_hbm,
                   r_a, c_a, v_a, r_b, c_b, v_b,
                   cnt0, cnt1, cnt2, cnt3, offs,
                   cur0, cur1, cur2, cur3, sidx_v, sval_v,
                   sem_a, sem_b):
        cnts_l = [cnt0, cnt1, cnt2, cnt3]
        curs_l = [cur0, cur1, cur2, cur3]
        c = lax.axis_index("c")
        s = lax.axis_index("s")
        w = s * NCORE + c
        my_off = w * share

        zero16 = jnp.zeros((16,), jnp.int32)

        @pl.loop(0, OTBL // 16)
        def _z(g):
            for u in range(NLANE):
                cnts_l[u][pl.ds(g * 16, 16)] = zero16

        def load_chunk(ci, r_v, c_v, v_v, sem, with_vals):
            off = my_off + ci * CHUNK
            pltpu.async_copy(rows_hbm.at[pl.ds(off, CHUNK)], r_v, sem)
            pltpu.async_copy(cols_hbm.at[pl.ds(off, CHUNK)], c_v, sem)
            if with_vals:
                pltpu.async_copy(vals_hbm.at[pl.ds(off, CHUNK)], v_v, sem)

        def wait_inputs(r_v, sem, n):
            for _ in range(n):
                pltpu.make_async_copy(
                    rows_hbm.at[pl.ds(0, CHUNK)], r_v, sem).wait()

        def count_chunk(r_v, c_v):
            @pl.loop(0, CHUNK // 64)
            def _b(it):
                for u in range(NLANE):
                    g = it * 64 + u * 16
                    rv = r_v[pl.ds(g, 16)]
                    cv = c_v[pl.ds(g, 16)]
                    t = lax.shift_right_logical(rv * HID + cv, SSH)
                    cnt, lastm = plsc.scan_count(t)
                    cur = plsc.load_gather(cnts_l[u], [t])
                    plsc.store_scatter(cnts_l[u], [t], cur + cnt,
                                       mask=lastm)

        def place_chunk(r_v, c_v, v_v):
            @pl.loop(0, CHUNK // 64)
            def _b(it):
                for u in range(NLANE):
                    g = it * 64 + u * 16
                    rv = r_v[pl.ds(g, 16)]
                    cv = c_v[pl.ds(g, 16)]
                    vv = v_v[pl.ds(g, 16)]
                    flat = rv * HID + cv
                    t = lax.shift_right_logical(flat, SSH)
                    lidx = flat & (SHARD - 1)
                    cnt, lastm = plsc.scan_count(t)
                    cur = plsc.load_gather(curs_l[u], [t])
                    pos = cur + cnt - 1
                    plsc.store_scatter(sidx_v, [pos], lidx)
                    plsc.store_scatter(sval_v, [pos], vv)
                    plsc.store_scatter(curs_l[u], [t], cur + cnt,
                                       mask=lastm)

        load_chunk(0, r_a, c_a, v_a, sem_a, False)
        load_chunk(1, r_b, c_b, v_b, sem_b, False)

        @pl.loop(0, n_pairs)
        def _pa(i):
            wait_inputs(r_a, sem_a, 2)
            count_chunk(r_a, c_a)

            @pl.when(i < n_pairs - 1)
            def _():
                load_chunk(2 * i + 2, r_a, c_a, v_a, sem_a, False)

            wait_inputs(r_b, sem_b, 2)
            count_chunk(r_b, c_b)

            @pl.when(i < n_pairs - 1)
            def _():
                load_chunk(2 * i + 3, r_b, c_b, v_b, sem_b, False)

        def pfx_body(g, carry):
            sl = pl.ds(g * 16, 16)
            c0 = cnt0[sl]
            c1 = cnt1[sl]
            c2 = cnt2[sl]
            c3 = cnt3[sl]
            tot = c0 + c1 + c2 + c3
            inc = plsc.cumsum(tot)
            ex = inc - tot + carry
            offs[sl] = ex
            cur0[sl] = ex
            cur1[sl] = ex + c0
            cur2[sl] = ex + c0 + c1
            cur3[sl] = ex + c0 + c1 + c2
            return carry + jnp.max(inc)

        lax.fori_loop(0, OTBL // 16, pfx_body, 0)
        pltpu.sync_copy(offs, offs_hbm.at[w])

        load_chunk(0, r_a, c_a, v_a, sem_a, True)
        load_chunk(1, r_b, c_b, v_b, sem_b, True)

        @pl.loop(0, n_pairs)
        def _pb(i):
            wait_inputs(r_a, sem_a, 3)
            place_chunk(r_a, c_a, v_a)

            @pl.when(i < n_pairs - 1)
            def _():
                load_chunk(2 * i + 2, r_a, c_a, v_a, sem_a, True)

            wait_inputs(r_b, sem_b, 3)
            place_chunk(r_b, c_b, v_b)

            @pl.when(i < n_pairs - 1)
            def _():
                load_chunk(2 * i + 3, r_b, c_b, v_b, sem_b, True)

        pltpu.sync_copy(sidx_v, sidx_hbm.at[w])
        pltpu.sync_copy(sval_v, sval_hbm.at[w])

    return bin_kernel(rows_p, cols_p, vals_p)


def _accum_phase(sidx, sval, offs):
    mesh, cp = _mesh_and_params()

    @functools.partial(
        pl.kernel,
        compiler_params=cp,
        out_type=jax.ShapeDtypeStruct((FS,), jnp.float32),
        mesh=mesh,
        scratch_types=[
            pltpu.VMEM((SHARD,), jnp.float32),
            pltpu.VMEM((NW, WIN), jnp.int32),
            pltpu.VMEM((NW, WIN), jnp.float32),
            pltpu.VMEM((NW, OTBL), jnp.int32),
            pltpu.SemaphoreType.DMA,
        ],
    )
    def accum_kernel(sidx_hbm, sval_hbm, offs_hbm, w_hbm,
                     acc, widx, wval, offs_all, sem):
        c = lax.axis_index("c")
        s = lax.axis_index("s")
        w = s * NCORE + c

        pltpu.sync_copy(offs_hbm, offs_all)
        zero16f = jnp.zeros((16,), jnp.float32)
        lanes = jnp.arange(16, dtype=jnp.int32)
        imin = jnp.int32(-(2 ** 31))

        def tbl_at(p, j):
            jal = pl.multiple_of(j & ~15, 8)
            vec = offs_all[p, pl.ds(jal, 16)]
            return jnp.max(jnp.where(lanes == j - jal, vec, imin))

        def seg_bounds(p, shc):
            return tbl_at(p, shc), tbl_at(p, shc + 1)

        @pl.loop(0, NGEN)
        def _gen(g):
            sh = g * NW + w
            shc = jnp.minimum(sh, OTBL - 2)

            @pl.loop(0, SHARD // 256)
            def _zb(j):
                for l in range(16):
                    acc[pl.ds(j * 256 + l * 16, 16)] = zero16f

            @pl.loop(0, NW)
            def _iss(p):
                st, _ = seg_bounds(p, shc)
                sal = pl.multiple_of(st & ~(WIN - 1), WIN)
                pltpu.async_copy(sidx_hbm.at[p, pl.ds(sal, WIN)],
                                 widx.at[p], sem)
                pltpu.async_copy(sval_hbm.at[p, pl.ds(sal, WIN)],
                                 wval.at[p], sem)

            @pl.loop(0, NW)
            def _drn(p):
                pltpu.make_async_copy(
                    sidx_hbm.at[0, pl.ds(0, WIN)], widx.at[p], sem).wait()
                pltpu.make_async_copy(
                    sval_hbm.at[0, pl.ds(0, WIN)], wval.at[p], sem).wait()

            @pl.loop(0, NW)
            def _proc(p):
                st, en = seg_bounds(p, shc)
                sal = pl.multiple_of(st & ~(WIN - 1), WIN)

                @pl.when(jnp.logical_and(en > st, sh < NSH))
                def _():
                    @pl.loop(0, WIN // 16)
                    def _g(gg):
                        posv = sal + gg * 16 + lanes
                        m = jnp.logical_and(posv >= st, posv < en)
                        lidx = widx[p, pl.ds(gg * 16, 16)]
                        vv = wval[p, pl.ds(gg * 16, 16)]
                        plsc.addupdate_scatter(acc, [lidx], vv, mask=m)

                    def rem_body(nxt):
                        wo = pl.multiple_of(nxt, WIN)
                        pltpu.sync_copy(
                            sidx_hbm.at[p, pl.ds(wo, WIN)],
                            widx.at[p])
                        pltpu.sync_copy(
                            sval_hbm.at[p, pl.ds(wo, WIN)],
                            wval.at[p])

                        @pl.loop(0, WIN // 16)
                        def _g2(gg):
                            posv = nxt + gg * 16 + lanes
                            m = posv < en
                            lidx = widx[p, pl.ds(gg * 16, 16)]
                            vv = wval[p, pl.ds(gg * 16, 16)]
                            plsc.addupdate_scatter(acc, [lidx], vv,
                                                   mask=m)

                        return nxt + WIN

                    lax.while_loop(lambda nxt: nxt < en, rem_body,
                                   sal + WIN)

            pltpu.sync_copy(
                acc, w_hbm.at[pl.ds(pl.multiple_of(sh * SHARD, 8), SHARD)])


    return accum_kernel(sidx, sval, offs)


def _mm_body(x_ref, w_ref, b_ref, o_ref):
    xb = x_ref[...]
    wb = w_ref[...].astype(jnp.bfloat16)
    acc = jnp.dot(xb, wb, preferred_element_type=jnp.float32)
    o_ref[...] = jnp.tanh(acc + b_ref[...])


def _matmul(xb, w, bias2d):
    batch = xb.shape[0]
    bn = 512
    return pl.pallas_call(
        _mm_body,
        grid=(HID // bn,),
        in_specs=[
            pl.BlockSpec((batch, IN_DIM), lambda j: (0, 0)),
            pl.BlockSpec((IN_DIM, bn), lambda j: (0, j)),
            pl.BlockSpec((1, bn), lambda j: (0, j)),
        ],
        out_specs=pl.BlockSpec((batch, bn), lambda j: (0, j)),
        out_shape=jax.ShapeDtypeStruct((batch, HID), jnp.float32),
    )(xb, w, bias2d)


def kernel(x, values, bias, rows, cols):
    nnz = rows.shape[0]
    grain = NW * 2 * CHUNK
    nnzp = -(-nnz // grain) * grain
    pad = nnzp - nnz
    rows_p = jnp.concatenate(
        [rows, jnp.full((pad,), IN_DIM, jnp.int32)])
    cols_p = jnp.concatenate([cols, jnp.zeros((pad,), jnp.int32)])
    vals_p = jnp.concatenate([values, jnp.zeros((pad,), jnp.float32)])

    sidx, sval, offs = _bin_phase(rows_p, cols_p, vals_p)
    w_flat = _accum_phase(sidx, sval, offs)
    w = w_flat.reshape(IN_DIM, HID)
    xb = x.astype(jnp.bfloat16)
    return _matmul(xb, w, bias.reshape(1, HID))


# speedup vs baseline: 1.3731x; 1.0133x over previous
import dataclasses
import functools

import jax
import jax.numpy as jnp
from jax import lax
from jax.experimental import pallas as pl
from jax.experimental.pallas import tpu as pltpu
from jax.experimental.pallas import tpu_sc as plsc

IN_DIM = 4096
HID = 4096
FS = IN_DIM * HID
NSUB = 16
NCORE = 2
NW = NSUB * NCORE
SHARD = 65536
SSH = 16
NSH = FS // SHARD
NGEN = NSH // NW
OTBL = 272
NLANE = 4
CHUNK = 1024
WIN = 512


def _mesh_and_params():
    mesh = plsc.VectorSubcoreMesh(core_axis_name="c", subcore_axis_name="s")
    cp = pltpu.CompilerParams()
    if "needs_layout_passes" in pltpu.CompilerParams.__dataclass_fields__:
        cp = dataclasses.replace(cp, needs_layout_passes=False)
    return mesh, cp


def _bin_phase(rows_p, cols_p, vals_p):
    nnzp = rows_p.shape[0]
    share = nnzp // NW
    n_pairs = share // (2 * CHUNK)
    scap = (share // WIN + 1) * WIN
    mesh, cp = _mesh_and_params()

    @functools.partial(
        pl.kernel,
        compiler_params=cp,
        out_type=(
            jax.ShapeDtypeStruct((NW, scap), jnp.int32),
            jax.ShapeDtypeStruct((NW, scap), jnp.float32),
            jax.ShapeDtypeStruct((NW, OTBL), jnp.int32),
        ),
        mesh=mesh,
        scratch_types=[
            pltpu.VMEM((CHUNK,), jnp.int32),
            pltpu.VMEM((CHUNK,), jnp.int32),
            pltpu.VMEM((CHUNK,), jnp.float32),
            pltpu.VMEM((CHUNK,), jnp.int32),
            pltpu.VMEM((CHUNK,), jnp.int32),
            pltpu.VMEM((CHUNK,), jnp.float32),
            pltpu.VMEM((OTBL,), jnp.int32),
            pltpu.VMEM((OTBL,), jnp.int32),
            pltpu.VMEM((OTBL,), jnp.int32),
            pltpu.VMEM((OTBL,), jnp.int32),
            pltpu.VMEM((OTBL,), jnp.int32),
            pltpu.VMEM((OTBL,), jnp.int32),
            pltpu.VMEM((OTBL,), jnp.int32),
            pltpu.VMEM((OTBL,), jnp.int32),
            pltpu.VMEM((OTBL,), jnp.int32),
            pltpu.VMEM((scap,), jnp.int32),
            pltpu.VMEM((scap,), jnp.float32),
            pltpu.SemaphoreType.DMA,
            pltpu.SemaphoreType.DMA,
        ],
    )
    def bin_kernel(rows_hbm, cols_hbm, vals_hbm,
                   sidx_hbm, sval_hbm, offs_hbm,
                   r_a, c_a, v_a, r_b, c_b, v_b,
                   cnt0, cnt1, cnt2, cnt3, offs,
                   cur0, cur1, cur2, cur3, sidx_v, sval_v,
                   sem_a, sem_b):
        cnts_l = [cnt0, cnt1, cnt2, cnt3]
        curs_l = [cur0, cur1, cur2, cur3]
        c = lax.axis_index("c")
        s = lax.axis_index("s")
        w = s * NCORE + c
        my_off = w * share

        zero16 = jnp.zeros((16,), jnp.int32)

        @pl.loop(0, OTBL // 16)
        def _z(g):
            for u in range(NLANE):
                cnts_l[u][pl.ds(g * 16, 16)] = zero16

        def load_chunk(ci, r_v, c_v, v_v, sem, with_vals):
            off = my_off + ci * CHUNK
            pltpu.async_copy(rows_hbm.at[pl.ds(off, CHUNK)], r_v, sem)
            pltpu.async_copy(cols_hbm.at[pl.ds(off, CHUNK)], c_v, sem)
            if with_vals:
                pltpu.async_copy(vals_hbm.at[pl.ds(off, CHUNK)], v_v, sem)

        def wait_inputs(r_v, sem, n):
            for _ in range(n):
                pltpu.make_async_copy(
                    rows_hbm.at[pl.ds(0, CHUNK)], r_v, sem).wait()

        def count_chunk(r_v, c_v):
            @pl.loop(0, CHUNK // 64)
            def _b(it):
                for u in range(NLANE):
                    g = it * 64 + u * 16
                    rv = r_v[pl.ds(g, 16)]
                    cv = c_v[pl.ds(g, 16)]
                    t = lax.shift_right_logical(rv * HID + cv, SSH)
                    cnt, lastm = plsc.scan_count(t)
                    cur = plsc.load_gather(cnts_l[u], [t])
                    plsc.store_scatter(cnts_l[u], [t], cur + cnt,
                                       mask=lastm)

        def place_chunk(r_v, c_v, v_v):
            @pl.loop(0, CHUNK // 64)
            def _b(it):
                for u in range(NLANE):
                    g = it * 64 + u * 16
                    rv = r_v[pl.ds(g, 16)]
                    cv = c_v[pl.ds(g, 16)]
                    vv = v_v[pl.ds(g, 16)]
                    flat = rv * HID + cv
                    t = lax.shift_right_logical(flat, SSH)
                    lidx = flat & (SHARD - 1)
                    cnt, lastm = plsc.scan_count(t)
                    cur = plsc.load_gather(curs_l[u], [t])
                    pos = cur + cnt - 1
                    plsc.store_scatter(sidx_v, [pos], lidx)
                    plsc.store_scatter(sval_v, [pos], vv)
                    plsc.store_scatter(curs_l[u], [t], cur + cnt,
                                       mask=lastm)

        load_chunk(0, r_a, c_a, v_a, sem_a, False)
        load_chunk(1, r_b, c_b, v_b, sem_b, False)

        @pl.loop(0, n_pairs)
        def _pa(i):
            wait_inputs(r_a, sem_a, 2)
            count_chunk(r_a, c_a)

            @pl.when(i < n_pairs - 1)
            def _():
                load_chunk(2 * i + 2, r_a, c_a, v_a, sem_a, False)

            wait_inputs(r_b, sem_b, 2)
            count_chunk(r_b, c_b)

            @pl.when(i < n_pairs - 1)
            def _():
                load_chunk(2 * i + 3, r_b, c_b, v_b, sem_b, False)

        def pfx_body(g, carry):
            sl = pl.ds(g * 16, 16)
            c0 = cnt0[sl]
            c1 = cnt1[sl]
            c2 = cnt2[sl]
            c3 = cnt3[sl]
            tot = c0 + c1 + c2 + c3
            inc = plsc.cumsum(tot)
            ex = inc - tot + carry
            offs[sl] = ex
            cur0[sl] = ex
            cur1[sl] = ex + c0
            cur2[sl] = ex + c0 + c1
            cur3[sl] = ex + c0 + c1 + c2
            return carry + jnp.max(inc)

        lax.fori_loop(0, OTBL // 16, pfx_body, 0)
        pltpu.sync_copy(offs, offs_hbm.at[w])

        load_chunk(0, r_a, c_a, v_a, sem_a, True)
        load_chunk(1, r_b, c_b, v_b, sem_b, True)

        @pl.loop(0, n_pairs)
        def _pb(i):
            wait_inputs(r_a, sem_a, 3)
            place_chunk(r_a, c_a, v_a)

            @pl.when(i < n_pairs - 1)
            def _():
                load_chunk(2 * i + 2, r_a, c_a, v_a, sem_a, True)

            wait_inputs(r_b, sem_b, 3)
            place_chunk(r_b, c_b, v_b)

            @pl.when(i < n_pairs - 1)
            def _():
                load_chunk(2 * i + 3, r_b, c_b, v_b, sem_b, True)

        pltpu.sync_copy(sidx_v, sidx_hbm.at[w])
        pltpu.sync_copy(sval_v, sval_hbm.at[w])

    return bin_kernel(rows_p, cols_p, vals_p)


def _accum_phase(sidx, sval, offs):
    mesh, cp = _mesh_and_params()

    @functools.partial(
        pl.kernel,
        compiler_params=cp,
        out_type=jax.ShapeDtypeStruct((FS,), jnp.float32),
        mesh=mesh,
        scratch_types=[
            pltpu.VMEM((SHARD,), jnp.float32),
            pltpu.VMEM((NW, WIN), jnp.int32),
            pltpu.VMEM((NW, WIN), jnp.float32),
            pltpu.VMEM((NW, OTBL), jnp.int32),
            pltpu.SemaphoreType.DMA,
        ],
    )
    def accum_kernel(sidx_hbm, sval_hbm, offs_hbm, w_hbm,
                     acc, widx, wval, offs_all, sem):
        c = lax.axis_index("c")
        s = lax.axis_index("s")
        w = s * NCORE + c

        pltpu.sync_copy(offs_hbm, offs_all)
        zero16f = jnp.zeros((16,), jnp.float32)
        lanes = jnp.arange(16, dtype=jnp.int32)
        imin = jnp.int32(-(2 ** 31))

        def tbl_at(p, j):
            jal = pl.multiple_of(j & ~15, 8)
            vec = offs_all[p, pl.ds(jal, 16)]
            return jnp.max(jnp.where(lanes == j - jal, vec, imin))

        def seg_bounds(p, shc):
            return tbl_at(p, shc), tbl_at(p, shc + 1)

        @pl.loop(0, NGEN)
        def _gen(g):
            sh = g * NW + w
            shc = jnp.minimum(sh, OTBL - 2)

            @pl.loop(0, NW)
            def _iss(p):
                st, _ = seg_bounds(p, shc)
                sal = pl.multiple_of(st & ~(WIN - 1), WIN)
                pltpu.async_copy(sidx_hbm.at[p, pl.ds(sal, WIN)],
                                 widx.at[p], sem)
                pltpu.async_copy(sval_hbm.at[p, pl.ds(sal, WIN)],
                                 wval.at[p], sem)

            @pl.loop(0, SHARD // 256)
            def _zb(j):
                for l in range(16):
                    acc[pl.ds(j * 256 + l * 16, 16)] = zero16f

            @pl.loop(0, NW)
            def _drn(p):
                pltpu.make_async_copy(
                    sidx_hbm.at[0, pl.ds(0, WIN)], widx.at[p], sem).wait()
                pltpu.make_async_copy(
                    sval_hbm.at[0, pl.ds(0, WIN)], wval.at[p], sem).wait()

            @pl.loop(0, NW)
            def _proc(p):
                st, en = seg_bounds(p, shc)
                sal = pl.multiple_of(st & ~(WIN - 1), WIN)

                @pl.when(jnp.logical_and(en > st, sh < NSH))
                def _():
                    @pl.loop(0, WIN // 16)
                    def _g(gg):
                        posv = sal + gg * 16 + lanes
                        m = jnp.logical_and(posv >= st, posv < en)
                        lidx = widx[p, pl.ds(gg * 16, 16)]
                        vv = wval[p, pl.ds(gg * 16, 16)]
                        plsc.addupdate_scatter(acc, [lidx], vv, mask=m)

                    def rem_body(nxt):
                        wo = pl.multiple_of(nxt, WIN)
                        pltpu.sync_copy(
                            sidx_hbm.at[p, pl.ds(wo, WIN)],
                            widx.at[p])
                        pltpu.sync_copy(
                            sval_hbm.at[p, pl.ds(wo, WIN)],
                            wval.at[p])

                        @pl.loop(0, WIN // 16)
                        def _g2(gg):
                            posv = nxt + gg * 16 + lanes
                            m = posv < en
                            lidx = widx[p, pl.ds(gg * 16, 16)]
                            vv = wval[p, pl.ds(gg * 16, 16)]
                            plsc.addupdate_scatter(acc, [lidx], vv,
                                                   mask=m)

                        return nxt + WIN

                    lax.while_loop(lambda nxt: nxt < en, rem_body,
                                   sal + WIN)

            pltpu.sync_copy(
                acc, w_hbm.at[pl.ds(pl.multiple_of(sh * SHARD, 8), SHARD)])


    return accum_kernel(sidx, sval, offs)


def _mm_body(x_ref, w_ref, b_ref, o_ref):
    xb = x_ref[...]
    wb = w_ref[...].astype(jnp.bfloat16)
    acc = jnp.dot(xb, wb, preferred_element_type=jnp.float32)
    o_ref[...] = jnp.tanh(acc + b_ref[...])


def _matmul(xb, w, bias2d):
    batch = xb.shape[0]
    bn = 512
    return pl.pallas_call(
        _mm_body,
        grid=(HID // bn,),
        in_specs=[
            pl.BlockSpec((batch, IN_DIM), lambda j: (0, 0)),
            pl.BlockSpec((IN_DIM, bn), lambda j: (0, j)),
            pl.BlockSpec((1, bn), lambda j: (0, j)),
        ],
        out_specs=pl.BlockSpec((batch, bn), lambda j: (0, j)),
        out_shape=jax.ShapeDtypeStruct((batch, HID), jnp.float32),
    )(xb, w, bias2d)


def kernel(x, values, bias, rows, cols):
    nnz = rows.shape[0]
    grain = NW * 2 * CHUNK
    nnzp = -(-nnz // grain) * grain
    pad = nnzp - nnz
    rows_p = jnp.concatenate(
        [rows, jnp.full((pad,), IN_DIM, jnp.int32)])
    cols_p = jnp.concatenate([cols, jnp.zeros((pad,), jnp.int32)])
    vals_p = jnp.concatenate([values, jnp.zeros((pad,), jnp.float32)])

    sidx, sval, offs = _bin_phase(rows_p, cols_p, vals_p)
    w_flat = _accum_phase(sidx, sval, offs)
    w = w_flat.reshape(IN_DIM, HID)
    xb = x.astype(jnp.bfloat16)
    return _matmul(xb, w, bias.reshape(1, HID))
